# baseline (speedup 1.0000x reference)
import functools

import jax
import jax.numpy as jnp
from jax import lax
from jax.experimental import pallas as pl
from jax.experimental.pallas import tpu as pltpu
from jax.experimental.pallas import tpu_sc as plsc

STEP_ = 3600.0
TABLE_LEN_ = 64
BATCH_ = 16384

_info = plsc.get_sparse_core_info()
_NC, _NS, _L = _info.num_cores, _info.num_subcores, _info.num_lanes
_NC = 1
_NW = _NC * _NS
_B_PER_W = BATCH_ // _NW
_UNROLL = 16


@functools.partial(
    pl.kernel,
    mesh=plsc.VectorSubcoreMesh(
        core_axis_name="c", subcore_axis_name="s", num_cores=_NC
    ),
    out_type=jax.ShapeDtypeStruct((BATCH_,), jnp.float32),
    scratch_types=[
        pltpu.VMEM((_B_PER_W,), jnp.float32),
        pltpu.VMEM((_B_PER_W,), jnp.float32),
        pltpu.VMEM((_B_PER_W,), jnp.float32),
        pltpu.VMEM((TABLE_LEN_,), jnp.float32),
        pltpu.SemaphoreType.DMA,
        pltpu.SemaphoreType.DMA,
        pltpu.SemaphoreType.DMA,
    ],
    compiler_params=pltpu.CompilerParams(needs_layout_passes=False),
)
def _file_pressure_sc(
    t_hbm, y_hbm, table_hbm, out_hbm, t_v, y_v, o_v, tab_v, s0, s1, s2
):
    wid = lax.axis_index("s") * _NC + lax.axis_index("c")
    base = wid * _B_PER_W
    cp_t = pltpu.async_copy(t_hbm.at[pl.ds(base, _B_PER_W)], t_v, s0)
    cp_y = pltpu.async_copy(y_hbm.at[pl.ds(base, _B_PER_W)], y_v, s1)
    cp_tab = pltpu.async_copy(table_hbm, tab_v, s2)
    cp_t.wait()
    cp_y.wait()
    cp_tab.wait()

    @plsc.parallel_loop(0, _B_PER_W // _L, unroll=_UNROLL)
    def _body(i):
        sl = pl.ds(i * _L, _L)
        idx = (t_v[sl] / STEP_).astype(jnp.int32)
        pressure = plsc.load_gather(tab_v, [idx])
        o_v[sl] = (pressure - y_v[sl]) / STEP_

    pltpu.sync_copy(o_v, out_hbm.at[pl.ds(base, _B_PER_W)])


@jax.jit
def kernel(t, y, table):
    return _file_pressure_sc(t, y, table)


# speedup vs baseline: 1.0107x; 1.0107x over previous
import functools

import jax
import jax.numpy as jnp
from jax import lax
from jax.experimental import pallas as pl
from jax.experimental.pallas import tpu as pltpu
from jax.experimental.pallas import tpu_sc as plsc

STEP_ = 3600.0
TABLE_LEN_ = 64
BATCH_ = 16384

_info = plsc.get_sparse_core_info()
_NC, _NS, _L = _info.num_cores, _info.num_subcores, _info.num_lanes
_NC = 1
_NW = _NC * _NS
_B_PER_W = BATCH_ // _NW
_UNROLL = 4


@functools.partial(
    pl.kernel,
    mesh=plsc.VectorSubcoreMesh(
        core_axis_name="c", subcore_axis_name="s", num_cores=_NC
    ),
    out_type=jax.ShapeDtypeStruct((BATCH_,), jnp.float32),
    scratch_types=[
        pltpu.VMEM((_B_PER_W,), jnp.float32),
        pltpu.VMEM((_B_PER_W,), jnp.float32),
        pltpu.VMEM((_B_PER_W,), jnp.float32),
        pltpu.VMEM((TABLE_LEN_,), jnp.float32),
        pltpu.SemaphoreType.DMA,
        pltpu.SemaphoreType.DMA,
        pltpu.SemaphoreType.DMA,
    ],
    compiler_params=pltpu.CompilerParams(needs_layout_passes=False),
)
def _file_pressure_sc(
    t_hbm, y_hbm, table_hbm, out_hbm, t_v, y_v, o_v, tab_v, s0, s1, s2
):
    wid = lax.axis_index("s") * _NC + lax.axis_index("c")
    base = wid * _B_PER_W
    cp_t = pltpu.async_copy(t_hbm.at[pl.ds(base, _B_PER_W)], t_v, s0)
    cp_y = pltpu.async_copy(y_hbm.at[pl.ds(base, _B_PER_W)], y_v, s1)
    cp_tab = pltpu.async_copy(table_hbm, tab_v, s2)
    cp_t.wait()
    cp_y.wait()
    cp_tab.wait()

    @plsc.parallel_loop(0, _B_PER_W // _L, unroll=_UNROLL)
    def _body(i):
        sl = pl.ds(i * _L, _L)
        idx = (t_v[sl] / STEP_).astype(jnp.int32)
        pressure = plsc.load_gather(tab_v, [idx])
        o_v[sl] = (pressure - y_v[sl]) / STEP_

    pltpu.sync_copy(o_v, out_hbm.at[pl.ds(base, _B_PER_W)])


@jax.jit
def kernel(t, y, table):
    return _file_pressure_sc(t, y, table)
